# baseline (speedup 1.0000x reference)
import jax
import jax.numpy as jnp
from jax import lax
from jax.experimental import pallas as pl
from jax.experimental.pallas import tpu as pltpu
from jax.experimental.pallas import tpu_sc as plsc

B = 1024
F = 26
V = 1000
D = 1024

NC = 2
NS = 16
NW = NC * NS
N = B * F
BW = B // NW
C = 16
NCH = F * BW // C
NBUF = 3
LANES = 16


def _body(table_hbm, xt_hbm, out_hbm, idx_v, bufs, shared, gsems, ssems,
          wsems):
    sid = lax.axis_index("s")
    wid = sid * NC + lax.axis_index("c")
    b0 = wid * BW

    blk = pl.multiple_of((wid // 4) * 128, 128)
    co = (wid % 4) * BW
    pltpu.sync_copy(xt_hbm.at[:, pl.ds(blk, 128)], idx_v)
    for f in range(F):
        for h in range(BW // LANES):
            sl = pl.ds(co + h * LANES, LANES)
            idx_v[f, sl] = idx_v[f, sl] + f * V

    def start_gather(f, h, b):
        pltpu.async_copy(table_hbm.at[idx_v.at[f, pl.ds(co + h * LANES, C)]],
                         bufs[b], gsems[b])

    def wait_gather(b):
        pltpu.make_async_copy(out_hbm.at[pl.ds(0, C)], bufs[b],
                              gsems[b]).wait()

    def start_stage(b):
        pltpu.async_copy(bufs[b], shared.at[sid, b], ssems[b])

    def wait_stage(b):
        pltpu.make_async_copy(bufs[b], shared.at[sid, b], ssems[b]).wait()

    def start_write(f, h, b):
        row = f * B + b0 + h * C
        pltpu.async_copy(shared.at[sid, b], out_hbm.at[pl.ds(row, C)],
                         wsems[b])

    def wait_write(b):
        pltpu.make_async_copy(shared.at[sid, b], out_hbm.at[pl.ds(0, C)],
                              wsems[b]).wait()

    start_gather(0, 0, 0)
    start_gather(0, 1, 1)
    wait_gather(0)
    start_stage(0)
    start_gather(1, 0, 2)
    wait_gather(1)
    start_stage(1)
    wait_stage(0)
    start_write(0, 0, 0)

    @pl.loop(NBUF, 51, step=NBUF)
    def _(c0):
        for k in range(NBUF):
            c = c0 + k
            f = lax.shift_right_logical(c, 1)
            h = c - f * 2
            fw = lax.shift_right_logical(c - 2, 1)
            hw = (c - 2) - fw * 2
            wait_write(k)
            start_gather(f, h, k)
            wait_gather((k + 2) % NBUF)
            start_stage((k + 2) % NBUF)
            wait_stage((k + 1) % NBUF)
            start_write(fw, hw, (k + 1) % NBUF)

    wait_write(0)
    start_gather(25, 1, 0)
    wait_gather(2)
    start_stage(2)
    wait_stage(1)
    start_write(24, 1, 1)
    wait_gather(0)
    start_stage(0)
    wait_stage(2)
    start_write(25, 0, 2)
    wait_stage(0)
    start_write(25, 1, 0)
    for k in range(NBUF):
        wait_write(k)


def _encode(table, xt):
    mesh = plsc.VectorSubcoreMesh(core_axis_name="c", subcore_axis_name="s")
    return pl.kernel(
        _body,
        out_type=jax.ShapeDtypeStruct((N, D), jnp.float32),
        mesh=mesh,
        scratch_types=[
            pltpu.VMEM((F, 128), jnp.int32),
            tuple(pltpu.VMEM((C, D), jnp.float32) for _ in range(NBUF)),
            pltpu.VMEM_SHARED((NS, NBUF, C, D), jnp.float32),
            tuple(pltpu.SemaphoreType.DMA for _ in range(NBUF)),
            tuple(pltpu.SemaphoreType.DMA for _ in range(NBUF)),
            tuple(pltpu.SemaphoreType.DMA for _ in range(NBUF)),
        ],
    )(table, xt)


def kernel(x, hv_matrix):
    xt = jnp.transpose(x).astype(jnp.int32)
    table = hv_matrix.reshape(F * V, D)
    out = _encode(table, xt)
    return jnp.transpose(out.reshape(F, B, D), (1, 0, 2))


# speedup vs baseline: 1.0222x; 1.0222x over previous
import jax
import jax.numpy as jnp
from jax import lax
from jax.experimental import pallas as pl
from jax.experimental.pallas import tpu as pltpu
from jax.experimental.pallas import tpu_sc as plsc

B = 1024
F = 26
V = 1000
D = 1024

NC = 2
NS = 16
NW = NC * NS
N = B * F
BW = B // NW
C = 16
NCH = F * BW // C
NBUF = 6
LAG = 3
LANES = 16


def _body(table_hbm, xt_hbm, out_hbm, idx_v, bufs, gsems, wsems):
    wid = lax.axis_index("s") * NC + lax.axis_index("c")
    b0 = wid * BW

    blk = pl.multiple_of((wid // 4) * 128, 128)
    co = (wid % 4) * BW
    pltpu.sync_copy(xt_hbm.at[:, pl.ds(blk, 128)], idx_v)
    for f in range(F):
        for h in range(BW // LANES):
            sl = pl.ds(co + h * LANES, LANES)
            idx_v[f, sl] = idx_v[f, sl] + f * V

    def start_gather(f, h, b):
        pltpu.async_copy(table_hbm.at[idx_v.at[f, pl.ds(co + h * LANES, C)]],
                         bufs[b], gsems[b])

    def wait_gather(b):
        pltpu.make_async_copy(out_hbm.at[pl.ds(0, C)], bufs[b],
                              gsems[b]).wait()

    def start_write(f, h, b):
        row = f * B + b0 + h * C
        pltpu.async_copy(bufs[b], out_hbm.at[pl.ds(row, C)], wsems[b])

    def wait_write(b):
        pltpu.make_async_copy(bufs[b], out_hbm.at[pl.ds(0, C)],
                              wsems[b]).wait()

    start_gather(0, 0, 0)
    start_gather(0, 1, 1)
    start_gather(1, 0, 2)
    wait_gather(0)
    start_write(0, 0, 0)
    start_gather(1, 1, 3)
    wait_gather(1)
    start_write(0, 1, 1)
    start_gather(2, 0, 4)
    wait_gather(2)
    start_write(1, 0, 2)
    start_gather(2, 1, 5)

    @pl.loop(NBUF, 48, step=NBUF)
    def _(c0):
        for k in range(NBUF):
            c = c0 + k
            fw = lax.shift_right_logical(c - LAG, 1)
            f = lax.shift_right_logical(c, 1)
            wait_gather((k + LAG) % NBUF)
            start_write(fw, (k - LAG) % 2, (k + LAG) % NBUF)
            wait_write(k)
            start_gather(f, k % 2, k)

    for c in range(48, 52):
        k = c % NBUF
        wait_gather((k + LAG) % NBUF)
        start_write((c - LAG) // 2, (c - LAG) % 2, (k + LAG) % NBUF)
        wait_write(k)
        start_gather(c // 2, c % 2, k)
    for c in range(52, 52 + LAG):
        k = c % NBUF
        wait_gather((k + LAG) % NBUF)
        start_write((c - LAG) // 2, (c - LAG) % 2, (k + LAG) % NBUF)
    for k in range(NBUF):
        wait_write(k)


def _encode(table, xt):
    mesh = plsc.VectorSubcoreMesh(core_axis_name="c", subcore_axis_name="s")
    return pl.kernel(
        _body,
        out_type=jax.ShapeDtypeStruct((N, D), jnp.float32),
        mesh=mesh,
        scratch_types=[
            pltpu.VMEM((F, 128), jnp.int32),
            tuple(pltpu.VMEM((C, D), jnp.float32) for _ in range(NBUF)),
            tuple(pltpu.SemaphoreType.DMA for _ in range(NBUF)),
            tuple(pltpu.SemaphoreType.DMA for _ in range(NBUF)),
        ],
    )(table, xt)


def kernel(x, hv_matrix):
    xt = jnp.transpose(x).astype(jnp.int32)
    table = hv_matrix.reshape(F * V, D)
    out = _encode(table, xt)
    return jnp.transpose(out.reshape(F, B, D), (1, 0, 2))
